# baseline (speedup 1.0000x reference)
import jax
import jax.numpy as jnp
from jax import lax
from jax.experimental import pallas as pl
from jax.experimental.pallas import tpu as pltpu

B, C, H, W = 32, 5, 384, 384
S = 2
RH = H // S


def _x_copy(x_hbm, xbuf, sem, b, h, slot):
  return pltpu.make_async_copy(
      x_hbm.at[b, pl.ds(1, C - 1), pl.ds(h * RH, RH)], xbuf.at[slot],
      sem.at[slot])


def _body(t_ref, x_hbm, out, xbuf, acc, cnt, sem):
  b = pl.program_id(0)
  h = pl.program_id(1)
  step = b * S + h
  slot = lax.rem(step, 2)
  nxt = 1 - slot

  @pl.when(step == 0)
  def _():
    _x_copy(x_hbm, xbuf, sem, 0, 0, 0).start()

  @pl.when(step + 1 < B * S)
  def _():
    nstep = step + 1
    _x_copy(x_hbm, xbuf, sem, nstep // S, lax.rem(nstep, S), nxt).start()

  _x_copy(x_hbm, xbuf, sem, b, h, slot).wait()

  mf = jnp.where(t_ref[0, 0] != 0.0, 1.0, 0.0)
  s = None
  for c in range(C - 1):
    d = t_ref[0, 1 + c] - xbuf[slot, c]
    s = d * d if s is None else s + d * d
  sq = s * mf

  @pl.when(step == 0)
  def _():
    acc[...] = sq
    cnt[...] = mf

  @pl.when(step > 0)
  def _():
    acc[...] += sq
    cnt[...] += mf

  @pl.when(step == B * S - 1)
  def _():
    out[0] = jnp.sum(acc[...])
    out[1] = jnp.sum(cnt[...])


def kernel(inputs, target):
  partial = pl.pallas_call(
      _body,
      grid=(B, S),
      in_specs=[
          pl.BlockSpec((1, C, RH, W), lambda b, h: (b, 0, h, 0)),
          pl.BlockSpec(memory_space=pl.ANY),
      ],
      out_specs=pl.BlockSpec(memory_space=pltpu.SMEM),
      out_shape=jax.ShapeDtypeStruct((2,), jnp.float32),
      scratch_shapes=[
          pltpu.VMEM((2, C - 1, RH, W), jnp.float32),
          pltpu.VMEM((RH, W), jnp.float32),
          pltpu.VMEM((RH, W), jnp.float32),
          pltpu.SemaphoreType.DMA((2,)),
      ],
  )(target, inputs)
  return partial[0] / (partial[1] * (C - 1))


# speedup vs baseline: 1.3492x; 1.3492x over previous
import jax
import jax.numpy as jnp
from jax import lax
from jax.experimental import pallas as pl
from jax.experimental.pallas import tpu as pltpu

B, C, H, W = 32, 5, 384, 384
G = 2
NSTEP = B // G


def _x_copy(x_hbm, xbuf, sem, g, slot):
  return pltpu.make_async_copy(
      x_hbm.at[pl.ds(g * G, G), pl.ds(1, C - 1)], xbuf.at[slot], sem.at[slot])


def _body(t_ref, x_hbm, out, xbuf, acc, cnt, sem):
  g = pl.program_id(0)
  slot = lax.rem(g, 2)
  nxt = 1 - slot

  @pl.when(g == 0)
  def _():
    _x_copy(x_hbm, xbuf, sem, 0, 0).start()

  @pl.when(g + 1 < NSTEP)
  def _():
    _x_copy(x_hbm, xbuf, sem, g + 1, nxt).start()

  _x_copy(x_hbm, xbuf, sem, g, slot).wait()

  sq = None
  mfs = None
  for i in range(G):
    mf = jnp.where(t_ref[i, 0] != 0.0, 1.0, 0.0)
    s = None
    for c in range(C - 1):
      d = t_ref[i, 1 + c] - xbuf[slot, i, c]
      s = d * d if s is None else s + d * d
    sm = s * mf
    sq = sm if sq is None else sq + sm
    mfs = mf if mfs is None else mfs + mf

  @pl.when(g == 0)
  def _():
    acc[...] = sq
    cnt[...] = mfs

  @pl.when(g > 0)
  def _():
    acc[...] += sq
    cnt[...] += mfs

  @pl.when(g == NSTEP - 1)
  def _():
    out[0] = jnp.sum(acc[...])
    out[1] = jnp.sum(cnt[...])


def kernel(inputs, target):
  partial = pl.pallas_call(
      _body,
      grid=(NSTEP,),
      in_specs=[
          pl.BlockSpec((G, C, H, W), lambda g: (g, 0, 0, 0)),
          pl.BlockSpec(memory_space=pl.ANY),
      ],
      out_specs=pl.BlockSpec(memory_space=pltpu.SMEM),
      out_shape=jax.ShapeDtypeStruct((2,), jnp.float32),
      scratch_shapes=[
          pltpu.VMEM((2, G, C - 1, H, W), jnp.float32),
          pltpu.VMEM((H, W), jnp.float32),
          pltpu.VMEM((H, W), jnp.float32),
          pltpu.SemaphoreType.DMA((2,)),
      ],
  )(target, inputs)
  return partial[0] / (partial[1] * (C - 1))
